# Initial kernel scaffold; baseline (speedup 1.0000x reference)
import jax
import jax.numpy as jnp
from jax.experimental import pallas as pl


def kernel(user_ids, item_ids, item_keywords, keyword_table, user_table, item_table, bias_u, bias_i, mlp_W, mlp_b):
    raise NotImplementedError("write your pallas kernel here")


# speedup vs baseline: 6.4253x; 6.4253x over previous
import jax
import jax.numpy as jnp
from jax import lax
from jax.experimental import pallas as pl
from jax.experimental.pallas import tpu as pltpu
from jax.experimental.pallas import tpu_sc as plsc

_NC = 2
_NS = 16
_NW = _NC * _NS
_L = 16
_B = 16384
_RPW = _B // _NW
_EMB = 64
_KWP = 20
_CH = 128
_NG = _RPW // _L


def _kw_score_body(kt_ref, w_ref, out_ref):
    w = w_ref[...][:, 0]
    out_ref[...] = jnp.sum(kt_ref[...] * w, axis=1)


def _kw_scores(keyword_table, mlp_W):
    n, d = keyword_table.shape
    blk = 8192
    grid = (n + blk - 1) // blk
    return pl.pallas_call(
        _kw_score_body,
        grid=(grid,),
        in_specs=[
            pl.BlockSpec((blk, d), lambda i: (i, 0)),
            pl.BlockSpec((d, 1), lambda i: (0, 0)),
        ],
        out_specs=pl.BlockSpec((blk,), lambda i: (i,)),
        out_shape=jax.ShapeDtypeStruct((n,), jnp.float32),
    )(keyword_table, mlp_W)


def _sc_body(uid_hbm, iid_hbm, kw16_hbm, utab_hbm, itab_hbm,
             bu_hbm, bi_hbm, kws_hbm, mb_hbm, out_hbm,
             uid_v, iid_v, u_rows, i_rows, ub_v, ib_v, widx_v, kwin_v,
             kwidx_t, kwval_v, out_v, mb_v,
             sem_ui, sem_b, sem_w, sem_kv):
    wid = lax.axis_index("s") * _NC + lax.axis_index("c")
    base = wid * _RPW
    pltpu.sync_copy(uid_hbm.at[pl.ds(base, _RPW)], uid_v)
    pltpu.sync_copy(iid_hbm.at[pl.ds(base, _RPW)], iid_v)
    pltpu.sync_copy(mb_hbm, mb_v.at[pl.ds(0, 1)])

    iota = lax.iota(jnp.int32, _L)
    zerosf = jnp.zeros((_L,), jnp.float32)

    def bw(g, carry):
        iid16 = iid_v[pl.ds(g * _L, _L)]
        a16 = lax.shift_right_logical(iid16 * _KWP, 4)
        pos = 2 * (g * _L + iota)
        plsc.store_scatter(widx_v, [pos], a16)
        plsc.store_scatter(widx_v, [pos + 1], a16 + 1)
        return carry

    lax.fori_loop(0, _NG, bw, 0)

    copies = []
    for c in range((2 * _RPW) // _CH):
        s = pl.ds(c * _CH, _CH)
        copies.append(pltpu.async_copy(kw16_hbm.at[widx_v.at[s]],
                                       kwin_v.at[s], sem_w))
    ui_copies = []
    for c in range(_RPW // _CH):
        s = pl.ds(c * _CH, _CH)
        ui_copies.append(pltpu.async_copy(utab_hbm.at[uid_v.at[s]],
                                          u_rows.at[s], sem_ui))
        ui_copies.append(pltpu.async_copy(itab_hbm.at[iid_v.at[s]],
                                          i_rows.at[s], sem_ui))
    b_copies = []
    for c in range(_RPW // _CH):
        s = pl.ds(c * _CH, _CH)
        b_copies.append(pltpu.async_copy(bu_hbm.at[uid_v.at[s]],
                                         ub_v.at[s], sem_b))
        b_copies.append(pltpu.async_copy(bi_hbm.at[iid_v.at[s]],
                                         ib_v.at[s], sem_b))
    for d in copies:
        d.wait()

    def tr(g, carry):
        iid16 = iid_v[pl.ds(g * _L, _L)]
        flat = iid16 * _KWP
        off = lax.bitwise_and(flat, 15)
        r32 = 32 * (g * _L + iota) + off
        for k in range(_KWP):
            p = r32 + k
            row = lax.shift_right_logical(p, 4)
            col = lax.bitwise_and(p, 15)
            vals = plsc.load_gather(kwin_v, [row, col])
            kwidx_t[pl.ds(k * _RPW + g * _L, _L)] = vals
        return carry

    lax.fori_loop(0, _NG, tr, 0)

    kv_copies = []
    for c in range((_RPW * _KWP) // _CH):
        s = pl.ds(c * _CH, _CH)
        kv_copies.append(pltpu.async_copy(kws_hbm.at[kwidx_t.at[s]],
                                          kwval_v.at[s], sem_kv))
    for d in ui_copies:
        d.wait()
    for d in b_copies:
        d.wait()

    mb = mb_v[...][0]

    def group(g, carry):
        acc = zerosf
        for l in range(_L):
            r = g * _L + l
            u0 = u_rows[r, pl.ds(0, _L)]
            u1 = u_rows[r, pl.ds(_L, _L)]
            u2 = u_rows[r, pl.ds(2 * _L, _L)]
            u3 = u_rows[r, pl.ds(3 * _L, _L)]
            i0 = i_rows[r, pl.ds(0, _L)]
            i1 = i_rows[r, pl.ds(_L, _L)]
            i2 = i_rows[r, pl.ds(2 * _L, _L)]
            i3 = i_rows[r, pl.ds(3 * _L, _L)]
            mf = u0 * i0 + u1 * i1 + u2 * i2 + u3 * i3
            s = jnp.sum(mf)
            acc = jnp.where(iota == l, s, acc)
        ub16 = ub_v[pl.ds(g * _L, _L)]
        ib16 = ib_v[pl.ds(g * _L, _L)]
        out_v[pl.ds(g * _L, _L)] = acc + ub16 + ib16 + mb
        return carry

    lax.fori_loop(0, _NG, group, 0)

    for d in kv_copies:
        d.wait()

    def kwadd(g, carry):
        acc = out_v[pl.ds(g * _L, _L)]
        for k in range(_KWP):
            acc = acc + kwval_v[pl.ds(k * _RPW + g * _L, _L)]
        out_v[pl.ds(g * _L, _L)] = acc
        return carry

    lax.fori_loop(0, _NG, kwadd, 0)
    pltpu.sync_copy(out_v, out_hbm.at[pl.ds(base, _RPW)])


def kernel(user_ids, item_ids, item_keywords, keyword_table, user_table,
           item_table, bias_u, bias_i, mlp_W, mlp_b):
    kw_score = _kw_scores(keyword_table, mlp_W)
    kw16 = jnp.reshape(item_keywords, (-1, 16))
    bu1 = jnp.reshape(bias_u, (-1,))
    bi1 = jnp.reshape(bias_i, (-1,))
    mesh = plsc.VectorSubcoreMesh(core_axis_name="c", subcore_axis_name="s",
                                  num_cores=_NC, num_subcores=_NS)
    run = pl.kernel(
        _sc_body,
        out_type=jax.ShapeDtypeStruct((_B,), jnp.float32),
        mesh=mesh,
        compiler_params=pltpu.CompilerParams(needs_layout_passes=False,
                                             use_tc_tiling_on_sc=False),
        scratch_types=[
            pltpu.VMEM((_RPW,), jnp.int32),
            pltpu.VMEM((_RPW,), jnp.int32),
            pltpu.VMEM((_RPW, _EMB), jnp.float32),
            pltpu.VMEM((_RPW, _EMB), jnp.float32),
            pltpu.VMEM((_RPW,), jnp.float32),
            pltpu.VMEM((_RPW,), jnp.float32),
            pltpu.VMEM((2 * _RPW,), jnp.int32),
            pltpu.VMEM((2 * _RPW, 16), jnp.int32),
            pltpu.VMEM((_RPW * _KWP,), jnp.int32),
            pltpu.VMEM((_RPW * _KWP,), jnp.float32),
            pltpu.VMEM((_RPW,), jnp.float32),
            pltpu.VMEM((_L,), jnp.float32),
            pltpu.SemaphoreType.DMA,
            pltpu.SemaphoreType.DMA,
            pltpu.SemaphoreType.DMA,
            pltpu.SemaphoreType.DMA,
        ],
    )
    return run(user_ids, item_ids, kw16, user_table, item_table,
               bu1, bi1, kw_score, mlp_b)
